# Initial kernel scaffold; baseline (speedup 1.0000x reference)
import jax
import jax.numpy as jnp
from jax.experimental import pallas as pl


def kernel(call_type_ids, contract_ids, func_selector_ids, depths, status_ids, input_sizes, output_sizes, gas_vals, trace_mask, type_table, contract_table, func_table, depth_table, status_table, W_in, b_in, W_out, b_out, W_gas, b_gas):
    raise NotImplementedError("write your pallas kernel here")


# speedup vs baseline: 1.7132x; 1.7132x over previous
import functools

import jax
import jax.numpy as jnp
from jax import lax
from jax.experimental import pallas as pl
from jax.experimental.pallas import tpu as pltpu
from jax.experimental.pallas import tpu_sc as plsc


def _make_sc_kernel(N, D, n_depth, n_status, NC, NS, C):
    NW = NC * NS
    RPW = N // NW
    NCH = RPW // C
    OUTW = 8 * D

    mesh = plsc.VectorSubcoreMesh(core_axis_name="c", subcore_axis_name="s")

    @functools.partial(
        pl.kernel,
        out_type=jax.ShapeDtypeStruct((N, OUTW), jnp.float32),
        mesh=mesh,
        compiler_params=pltpu.CompilerParams(use_tc_tiling_on_sc=False),
        scratch_types=[
            pltpu.VMEM((6, C), jnp.int32),
            pltpu.VMEM((3, C), jnp.float32),
            pltpu.VMEM((6, D), jnp.float32),
            pltpu.VMEM((C, D), jnp.float32),
            pltpu.VMEM((C, D), jnp.float32),
            pltpu.VMEM((C, D), jnp.float32),
            pltpu.VMEM((C, D), jnp.float32),
            pltpu.VMEM((C, D), jnp.float32),
            pltpu.VMEM((C, 3 * D), jnp.float32),
            pltpu.SemaphoreType.DMA,
            pltpu.SemaphoreType.DMA,
        ],
    )
    def k(ids_hbm, scal_hbm, wb_hbm, type_hbm, contract_hbm, func_hbm,
          depth_hbm, status_hbm, out_hbm,
          idx_v, scal_v, wb_v, gt_v, gc_v, gf_v, gd_v, gs_v, proj_v,
          sem_g, sem_o):
        wid = lax.axis_index("s") * NC + lax.axis_index("c")
        pltpu.sync_copy(wb_hbm, wb_v)

        def chunk(i, carry):
            base = wid * RPW + i * C
            pltpu.sync_copy(ids_hbm.at[:, pl.ds(base, C)], idx_v)
            pltpu.sync_copy(scal_hbm.at[:, pl.ds(base, C)], scal_v)
            cp1 = pltpu.async_copy(type_hbm.at[idx_v.at[0]], gt_v, sem_g)
            cp2 = pltpu.async_copy(contract_hbm.at[idx_v.at[1]], gc_v, sem_g)
            cp3 = pltpu.async_copy(func_hbm.at[idx_v.at[2]], gf_v, sem_g)
            for g in range(C // 16):
                sl = pl.ds(g * 16, 16)
                m = idx_v[5, sl]
                dep = jnp.minimum(jnp.maximum(idx_v[3, sl], 0), n_depth - 2) + 1
                st = jnp.minimum(jnp.maximum(idx_v[4, sl], 0), n_status - 2) + 1
                zero = jnp.zeros((16,), jnp.int32)
                idx_v[3, sl] = jnp.where(m != 0, dep, zero)
                idx_v[4, sl] = jnp.where(m != 0, st, zero)
            cp4 = pltpu.async_copy(depth_hbm.at[idx_v.at[3]], gd_v, sem_g)
            cp5 = pltpu.async_copy(status_hbm.at[idx_v.at[4]], gs_v, sem_g)
            wlo = [wb_v[2 * f, pl.ds(0, 16)] for f in range(3)]
            whi = [wb_v[2 * f, pl.ds(16, 16)] for f in range(3)]
            blo = [wb_v[2 * f + 1, pl.ds(0, 16)] for f in range(3)]
            bhi = [wb_v[2 * f + 1, pl.ds(16, 16)] for f in range(3)]

            def pgroup(g, c2):
                xvs = [scal_v[f, pl.ds(g * 16, 16)] for f in range(3)]
                for r in range(16):
                    row = g * 16 + r
                    for f in range(3):
                        x = xvs[f][r]
                        lo = jnp.maximum(x * wlo[f] + blo[f], 0.0)
                        hi = jnp.maximum(x * whi[f] + bhi[f], 0.0)
                        proj_v[row, pl.ds(f * 32, 16)] = lo
                        proj_v[row, pl.ds(f * 32 + 16, 16)] = hi
                return c2

            lax.fori_loop(0, C // 16, pgroup, 0)
            cp1.wait(); cp2.wait(); cp3.wait(); cp4.wait(); cp5.wait()
            w1 = pltpu.async_copy(gt_v, out_hbm.at[pl.ds(base, C), pl.ds(0, D)], sem_o)
            w2 = pltpu.async_copy(gc_v, out_hbm.at[pl.ds(base, C), pl.ds(D, D)], sem_o)
            w3 = pltpu.async_copy(gf_v, out_hbm.at[pl.ds(base, C), pl.ds(2 * D, D)], sem_o)
            w4 = pltpu.async_copy(gd_v, out_hbm.at[pl.ds(base, C), pl.ds(3 * D, D)], sem_o)
            w5 = pltpu.async_copy(gs_v, out_hbm.at[pl.ds(base, C), pl.ds(4 * D, D)], sem_o)
            w6 = pltpu.async_copy(proj_v, out_hbm.at[pl.ds(base, C), pl.ds(5 * D, 3 * D)], sem_o)
            w1.wait(); w2.wait(); w3.wait(); w4.wait(); w5.wait(); w6.wait()
            return carry

        lax.fori_loop(0, NCH, chunk, 0)

    return k


def kernel(call_type_ids, contract_ids, func_selector_ids, depths, status_ids,
           input_sizes, output_sizes, gas_vals, trace_mask,
           type_table, contract_table, func_table, depth_table, status_table,
           W_in, b_in, W_out, b_out, W_gas, b_gas):
    B, L = call_type_ids.shape
    D = type_table.shape[1]
    N = B * L
    ids = jnp.stack([
        call_type_ids.reshape(N), contract_ids.reshape(N),
        func_selector_ids.reshape(N), depths.reshape(N),
        status_ids.reshape(N), trace_mask.reshape(N).astype(jnp.int32),
    ]).astype(jnp.int32)
    scal = jnp.stack([input_sizes.reshape(N), output_sizes.reshape(N),
                      gas_vals.reshape(N)]).astype(jnp.float32)
    wb = jnp.stack([W_in[:, 0], b_in, W_out[:, 0], b_out, W_gas[:, 0], b_gas])
    info = plsc.get_sparse_core_info()
    k = _make_sc_kernel(N, D, depth_table.shape[0], status_table.shape[0],
                        info.num_cores, info.num_subcores, 128)
    out = k(ids, scal, wb, type_table, contract_table, func_table,
            depth_table, status_table)
    return out.reshape(B, L, 8 * D)


# speedup vs baseline: 7.2681x; 4.2424x over previous
import functools

import jax
import jax.numpy as jnp
from jax import lax
from jax.experimental import pallas as pl
from jax.experimental.pallas import tpu as pltpu
from jax.experimental.pallas import tpu_sc as plsc


def _make_sc_kernel(N, D, n_type, n_depth, n_status, NC, NS, C):
    n_small = n_type + n_depth + n_status
    d_off = n_type
    s_off = n_type + n_depth
    NW = NC * NS
    RPW = N // NW
    NCH = RPW // C
    NH = NCH // 2
    OUTW = 8 * D

    mesh = plsc.VectorSubcoreMesh(core_axis_name="c", subcore_axis_name="s")

    def buf2(shape, dtype):
        return [pltpu.VMEM(shape, dtype), pltpu.VMEM(shape, dtype)]

    @functools.partial(
        pl.kernel,
        out_type=jax.ShapeDtypeStruct((N, OUTW), jnp.float32),
        mesh=mesh,
        compiler_params=pltpu.CompilerParams(use_tc_tiling_on_sc=False,
                                             needs_layout_passes=False),
        scratch_types=[
            buf2((6, C), jnp.int32),
            buf2((3, C), jnp.float32),
            pltpu.VMEM((6, D), jnp.float32),
            pltpu.VMEM((n_small * D,), jnp.float32),
            buf2((C, D), jnp.float32),
            buf2((C, D), jnp.float32),
            buf2((C, D), jnp.float32),
            buf2((C, D), jnp.float32),
            buf2((C, D), jnp.float32),
            buf2((C, 3 * D), jnp.float32),
            [pltpu.SemaphoreType.DMA] * 2,
            [pltpu.SemaphoreType.DMA] * 2,
            [pltpu.SemaphoreType.DMA] * 2,
        ],
    )
    def k(ids_hbm, scal_hbm, wb_hbm, small_hbm, contract_hbm, func_hbm,
          out_hbm,
          idx_v, scal_v, wb_v, stbl_v, gt_v, gc_v, gf_v, gd_v, gs_v, proj_v,
          sem_s, sem_g, sem_o):
        wid = lax.axis_index("s") * NC + lax.axis_index("c")
        row0 = wid * RPW
        pltpu.sync_copy(wb_hbm, wb_v)
        pltpu.sync_copy(small_hbm, stbl_v)

        def fire_stage(base, b):
            pltpu.async_copy(ids_hbm.at[:, pl.ds(base, C)], idx_v[b], sem_s[b])
            pltpu.async_copy(scal_hbm.at[:, pl.ds(base, C)], scal_v[b], sem_s[b])

        def wait_stage(base, b):
            pltpu.make_async_copy(ids_hbm.at[:, pl.ds(base, C)], idx_v[b], sem_s[b]).wait()
            pltpu.make_async_copy(scal_hbm.at[:, pl.ds(base, C)], scal_v[b], sem_s[b]).wait()

        def out_slices(base):
            return [out_hbm.at[pl.ds(base, C), pl.ds(f * D, D)] for f in range(5)] + \
                   [out_hbm.at[pl.ds(base, C), pl.ds(5 * D, 3 * D)]]

        def bufs(b):
            return [gt_v[b], gc_v[b], gf_v[b], gd_v[b], gs_v[b], proj_v[b]]

        def drain_writes(base, b):
            for src, dst in zip(bufs(b), out_slices(base)):
                pltpu.make_async_copy(src, dst, sem_o[b]).wait()

        def fire_writes(base, b):
            for src, dst in zip(bufs(b), out_slices(base)):
                pltpu.async_copy(src, dst, sem_o[b])

        iota16 = lax.iota(jnp.int32, 16)

        fire_stage(row0, 0)

        def outer(i, carry):
            for b in range(2):
                base = row0 + (2 * i + b) * C
                if b == 0:
                    fire_stage(base + C, 1)
                else:
                    @pl.when(i < NH - 1)
                    def _():
                        fire_stage(base + C, 0)
                wait_stage(base, b)
                @pl.when(i > 0)
                def _():
                    drain_writes(base, b)
                cp2 = pltpu.async_copy(contract_hbm.at[idx_v[b].at[1]], gc_v[b], sem_g[b])
                cp3 = pltpu.async_copy(func_hbm.at[idx_v[b].at[2]], gf_v[b], sem_g[b])
                wlo = [wb_v[2 * f, pl.ds(0, 16)] for f in range(3)]
                whi = [wb_v[2 * f, pl.ds(16, 16)] for f in range(3)]
                blo = [wb_v[2 * f + 1, pl.ds(0, 16)] for f in range(3)]
                bhi = [wb_v[2 * f + 1, pl.ds(16, 16)] for f in range(3)]

                def pgroup(g, c2):
                    sl = pl.ds(g * 16, 16)
                    xvs = [scal_v[b][f, sl] for f in range(3)]
                    m = idx_v[b][5, sl]
                    tvec = idx_v[b][0, sl] * D
                    dep = jnp.minimum(jnp.maximum(idx_v[b][3, sl], 0), n_depth - 2) + 1
                    st = jnp.minimum(jnp.maximum(idx_v[b][4, sl], 0), n_status - 2) + 1
                    zero = jnp.zeros((16,), jnp.int32)
                    dvec = (jnp.where(m != 0, dep, zero) + d_off) * D
                    svec = (jnp.where(m != 0, st, zero) + s_off) * D
                    for r in range(16):
                        row = g * 16 + r
                        for ids, dst in ((tvec, gt_v), (dvec, gd_v), (svec, gs_v)):
                            a0 = ids[r] + iota16
                            dst[b][row, pl.ds(0, 16)] = plsc.load_gather(stbl_v, [a0])
                            dst[b][row, pl.ds(16, 16)] = plsc.load_gather(stbl_v, [a0 + 16])
                        for f in range(3):
                            x = xvs[f][r]
                            lo = jnp.maximum(x * wlo[f] + blo[f], 0.0)
                            hi = jnp.maximum(x * whi[f] + bhi[f], 0.0)
                            proj_v[b][row, pl.ds(f * 32, 16)] = lo
                            proj_v[b][row, pl.ds(f * 32 + 16, 16)] = hi
                    return c2

                lax.fori_loop(0, C // 16, pgroup, 0)
                cp2.wait(); cp3.wait()
                fire_writes(base, b)
            return carry

        lax.fori_loop(0, NH, outer, 0)
        drain_writes(row0, 0)
        drain_writes(row0, 1)

    return k


def kernel(call_type_ids, contract_ids, func_selector_ids, depths, status_ids,
           input_sizes, output_sizes, gas_vals, trace_mask,
           type_table, contract_table, func_table, depth_table, status_table,
           W_in, b_in, W_out, b_out, W_gas, b_gas):
    B, L = call_type_ids.shape
    D = type_table.shape[1]
    N = B * L
    ids = jnp.stack([
        call_type_ids.reshape(N), contract_ids.reshape(N),
        func_selector_ids.reshape(N), depths.reshape(N),
        status_ids.reshape(N), trace_mask.reshape(N).astype(jnp.int32),
    ]).astype(jnp.int32)
    scal = jnp.stack([input_sizes.reshape(N), output_sizes.reshape(N),
                      gas_vals.reshape(N)]).astype(jnp.float32)
    wb = jnp.stack([W_in[:, 0], b_in, W_out[:, 0], b_out, W_gas[:, 0], b_gas])
    small = jnp.concatenate([type_table, depth_table, status_table],
                            axis=0).reshape(-1)
    info = plsc.get_sparse_core_info()
    k = _make_sc_kernel(N, D, type_table.shape[0], depth_table.shape[0],
                        status_table.shape[0], info.num_cores,
                        info.num_subcores, 128)
    out = k(ids, scal, wb, small, contract_table, func_table)
    return out.reshape(B, L, 8 * D)
